# Initial kernel scaffold; baseline (speedup 1.0000x reference)
import jax
import jax.numpy as jnp
from jax.experimental import pallas as pl


def kernel(context, targets, embedding):
    raise NotImplementedError("write your pallas kernel here")


# speedup vs baseline: 2.8529x; 2.8529x over previous
import functools

import jax
import jax.numpy as jnp
from jax import lax
from jax.experimental import pallas as pl
from jax.experimental.pallas import tpu as pltpu
from jax.experimental.pallas import tpu_sc as plsc

NC = 2
NS = 16
NW = NC * NS

B = 16384
L = 20
T = 5
D = 64
LANES = 16
DV = D // LANES

ROWS_PER_W = B // NW
SB = 32
NSUB = ROWS_PER_W // SB
CTX_I = SB * L
TGT_I = SB * T


def _cbow_body(ctx_hbm, tgt_hbm, emb_hbm, out_hbm,
               ctx_idx, tgt_idx, ctx_rows, tgt_rows, acc_t, out_tile,
               sem_g0, sem_g1):
  wid = lax.axis_index("s") * NC + lax.axis_index("c")
  gsems = (sem_g0, sem_g1)

  def copy_indices(s, nb):
    r0 = wid * (NSUB * 8) + s * 8
    pltpu.sync_copy(ctx_hbm.at[pl.ds(r0, 8)], ctx_idx.at[nb])
    pltpu.sync_copy(tgt_hbm.at[pl.ds(r0, 8)], tgt_idx.at[nb])

  def fire_gathers(nb):
    sem = gsems[nb]
    ds = []
    for i in range(8):
      ds.append(pltpu.async_copy(
          emb_hbm.at[ctx_idx.at[nb, i]],
          ctx_rows.at[nb].at[pl.ds(i * 80, 80)], sem))
    for i in range(8):
      ds.append(pltpu.async_copy(
          emb_hbm.at[tgt_idx.at[nb, i]],
          tgt_rows.at[nb].at[pl.ds(i * 20, 20)], sem))
    return ds

  def compute(s, buf):
    crows = ctx_rows.at[buf]
    trows = tgt_rows.at[buf]
    lanes = lax.iota(jnp.int32, LANES)

    def body(b, carry):
      cb = b * L
      vc = [crows[cb, pl.ds(k * LANES, LANES)] for k in range(DV)]
      for j in range(1, L):
        for k in range(DV):
          vc[k] = vc[k] + crows[cb + j, pl.ds(k * LANES, LANES)]
      scale = jnp.float32(1.0 / L)
      vc = [v * scale for v in vc]
      tb = b * T
      for t in range(T):
        acc = vc[0] * trows[tb + t, pl.ds(0, LANES)]
        for k in range(1, DV):
          acc = acc + vc[k] * trows[tb + t, pl.ds(k * LANES, LANES)]
        pair = jnp.full((LANES,), tb + t, dtype=jnp.int32)
        plsc.store_scatter(acc_t, [lanes, pair], acc)
      return carry

    lax.fori_loop(0, SB, body, 0, unroll=1)

    def rbody(g, carry):
      p0 = g * LANES
      tot = acc_t[0, pl.ds(p0, LANES)]
      for l in range(1, LANES):
        tot = tot + acc_t[l, pl.ds(p0, LANES)]
      out_tile[pl.ds(p0, LANES)] = tot
      return carry

    lax.fori_loop(0, (SB * T) // LANES, rbody, 0, unroll=1)
    e0 = (wid * ROWS_PER_W + s * SB) * T
    pltpu.sync_copy(out_tile, out_hbm.at[pl.ds(e0, SB * T)])

  copy_indices(0, 0)
  pending = fire_gathers(0)
  for s in range(NSUB):
    buf = s & 1
    nb = (s + 1) & 1
    if s + 1 < NSUB:
      copy_indices(s + 1, nb)
      nxt = fire_gathers(nb)
    else:
      nxt = []
    for d in pending:
      d.wait()
    compute(s, buf)
    pending = nxt


@jax.jit
def kernel(context, targets, embedding):
  ctx2 = context.astype(jnp.int32).reshape(-1, 80)
  tgt2 = targets.astype(jnp.int32).reshape(-1, 20)

  run = functools.partial(
      pl.kernel,
      out_type=jax.ShapeDtypeStruct((B * T,), jnp.float32),
      mesh=plsc.VectorSubcoreMesh(core_axis_name="c", subcore_axis_name="s"),
      compiler_params=pltpu.CompilerParams(
          needs_layout_passes=False, use_tc_tiling_on_sc=False),
      scratch_types=[
          pltpu.VMEM((2, 8, 80), jnp.int32),
          pltpu.VMEM((2, 8, 20), jnp.int32),
          pltpu.VMEM((2, CTX_I, D), jnp.float32),
          pltpu.VMEM((2, TGT_I, D), jnp.float32),
          pltpu.VMEM((LANES, SB * T), jnp.float32),
          pltpu.VMEM((SB * T,), jnp.float32),
          pltpu.SemaphoreType.DMA,
          pltpu.SemaphoreType.DMA,
      ],
  )(_cbow_body)
  return run(ctx2, tgt2, embedding).reshape(B, T)


# speedup vs baseline: 3.1461x; 1.1028x over previous
import functools

import jax
import jax.numpy as jnp
from jax import lax
from jax.experimental import pallas as pl
from jax.experimental.pallas import tpu as pltpu
from jax.experimental.pallas import tpu_sc as plsc

NC = 2
NS = 16
NW = NC * NS

B = 16384
L = 20
T = 5
D = 64
W = 128
LANES = 16
DV = D // LANES

RPW = B // NW
SB = 16
NSUB = RPW // SB
CI = SB * L
TI = SB * T


def _cbow_body(ctx_hbm, tgt_hbm, emb_hbm, out_hbm,
               ctx_idx, tgt_idx, ctx_rows, tgt_rows, acc_t, out_tile,
               sem_g0, sem_g1):
  wid = lax.axis_index("s") * NC + lax.axis_index("c")
  gsems = (sem_g0, sem_g1)

  pltpu.sync_copy(ctx_hbm.at[pl.ds(wid * (RPW * L), RPW * L)], ctx_idx)
  pltpu.sync_copy(tgt_hbm.at[pl.ds(wid * (RPW * T), RPW * T)], tgt_idx)

  def gather_list(s, nb):
    c0 = s * CI
    t0 = s * TI
    return [
        (ctx_idx.at[pl.ds(c0, 128)], ctx_rows.at[nb].at[pl.ds(0, 128)]),
        (ctx_idx.at[pl.ds(c0 + 128, 128)], ctx_rows.at[nb].at[pl.ds(128, 128)]),
        (ctx_idx.at[pl.ds(c0 + 256, 64)], ctx_rows.at[nb].at[pl.ds(256, 64)]),
        (tgt_idx.at[pl.ds(t0, TI)], tgt_rows.at[nb]),
    ]

  def fire(s, nb):
    for idx, dst in gather_list(s, nb):
      pltpu.async_copy(emb_hbm.at[idx], dst, gsems[nb])

  def drain(s, buf):
    for idx, dst in gather_list(s, buf):
      pltpu.make_async_copy(emb_hbm.at[idx], dst, gsems[buf]).wait()

  def compute(s, buf):
    crows = ctx_rows.at[buf]
    trows = tgt_rows.at[buf]
    lanes = lax.iota(jnp.int32, LANES)

    def body(b, carry):
      cb = b * L
      vc = [crows[cb, pl.ds(k * LANES, LANES)] for k in range(DV)]
      for j in range(1, L):
        for k in range(DV):
          vc[k] = vc[k] + crows[cb + j, pl.ds(k * LANES, LANES)]
      scale = jnp.float32(1.0 / L)
      vc = [v * scale for v in vc]
      tb = b * T
      for t in range(T):
        acc = vc[0] * trows[tb + t, pl.ds(0, LANES)]
        for k in range(1, DV):
          acc = acc + vc[k] * trows[tb + t, pl.ds(k * LANES, LANES)]
        pair = jnp.full((LANES,), tb + t, dtype=jnp.int32)
        plsc.store_scatter(acc_t, [lanes, pair], acc)
      return carry

    lax.fori_loop(0, SB, body, 0, unroll=1)

    for g in range(TI // LANES):
      p0 = g * LANES
      tot = acc_t[0, pl.ds(p0, LANES)]
      for l in range(1, LANES):
        tot = tot + acc_t[l, pl.ds(p0, LANES)]
      out_tile[pl.ds(p0, LANES)] = tot

    e0 = (wid * RPW + s * SB) * T
    pltpu.sync_copy(out_tile, out_hbm.at[pl.ds(e0, TI)])

  fire(0, 0)

  def outer(m, carry):
    s = m * 2
    fire(s + 1, 1)
    drain(s, 0)
    compute(s, 0)
    fire(s + 2, 0)
    drain(s + 1, 1)
    compute(s + 1, 1)
    return carry

  lax.fori_loop(0, NSUB // 2 - 1, outer, 0, unroll=1)
  s = NSUB - 2
  fire(s + 1, 1)
  drain(s, 0)
  compute(s, 0)
  drain(s + 1, 1)
  compute(s + 1, 1)


@jax.jit
def kernel(context, targets, embedding):
  ctx_flat = context.astype(jnp.int32).reshape(-1)
  tgt_flat = targets.astype(jnp.int32).reshape(-1)
  emb_pad = jnp.pad(embedding, ((0, 0), (0, W - D)))

  run = functools.partial(
      pl.kernel,
      out_type=jax.ShapeDtypeStruct((B * T,), jnp.float32),
      mesh=plsc.VectorSubcoreMesh(core_axis_name="c", subcore_axis_name="s"),
      compiler_params=pltpu.CompilerParams(
          needs_layout_passes=False, use_tc_tiling_on_sc=True),
      scratch_types=[
          pltpu.VMEM((RPW * L,), jnp.int32),
          pltpu.VMEM((RPW * T,), jnp.int32),
          pltpu.VMEM((2, CI, W), jnp.float32),
          pltpu.VMEM((2, TI, W), jnp.float32),
          pltpu.VMEM((LANES, TI), jnp.float32),
          pltpu.VMEM((TI,), jnp.float32),
          pltpu.SemaphoreType.DMA,
          pltpu.SemaphoreType.DMA,
      ],
  )(_cbow_body)
  return run(ctx_flat, tgt_flat, emb_pad).reshape(B, T)
